# baseline (speedup 1.0000x reference)
import jax
import jax.numpy as jnp
from jax import lax
from jax.experimental import pallas as pl
from jax.experimental.pallas import tpu as pltpu
from jax.experimental.pallas import tpu_sc as plsc

B = 4096
WL = 5
D = 64
H = 128
T = 50
NI = B * WL

NC, NS = 2, 16
NW = NC * NS
ROWS_W = B // NW
PER_W = NI // NW

BB = 512


def _idx_prep_body(wi, si, pi, wo, so, po):
    wo[...] = wi[...].T
    so[...] = si[...].T
    po[...] = pi[...].T


def _idx_prep(words, suffix, prefix):
    return pl.pallas_call(
        _idx_prep_body,
        out_shape=[jax.ShapeDtypeStruct((WL, B), jnp.int32)] * 3,
    )(words, suffix, prefix)


def _sc_gather_body(ew, ep, es, wi, pi, si, ow, op_, os_,
                    wv, pv, sv, rw, rp, rs, sem):
    wid = lax.axis_index("s") * NC + lax.axis_index("c")
    b0 = wid * ROWS_W
    csl = (slice(None), pl.ds(b0, ROWS_W))
    idx_cps = [
        pltpu.async_copy(wi.at[csl], wv, sem),
        pltpu.async_copy(pi.at[csl], pv, sem),
        pltpu.async_copy(si.at[csl], sv, sem),
    ]
    for cp in idx_cps:
        cp.wait()
    gather_cps = []
    for w in range(WL):
        rsl = pl.ds(w * ROWS_W, ROWS_W)
        gather_cps.append(pltpu.async_copy(ew.at[wv.at[w]], rw.at[rsl], sem))
        gather_cps.append(pltpu.async_copy(ep.at[pv.at[w]], rp.at[rsl], sem))
        gather_cps.append(pltpu.async_copy(es.at[sv.at[w]], rs.at[rsl], sem))
    for cp in gather_cps:
        cp.wait()
    out_cps = []
    for w in range(WL):
        rsl = pl.ds(w * ROWS_W, ROWS_W)
        osl = pl.ds(w * B + b0, ROWS_W)
        out_cps.append(pltpu.async_copy(rw.at[rsl], ow.at[osl], sem))
        out_cps.append(pltpu.async_copy(rp.at[rsl], op_.at[osl], sem))
        out_cps.append(pltpu.async_copy(rs.at[rsl], os_.at[osl], sem))
    for cp in out_cps:
        cp.wait()


def _sc_gather(emb_word, emb_pref, emb_suff, widx, pidx, sidx):
    mesh = plsc.VectorSubcoreMesh(core_axis_name="c", subcore_axis_name="s")
    out_t = [jax.ShapeDtypeStruct((NI, D), jnp.float32)] * 3
    scratch = [
        pltpu.VMEM((WL, ROWS_W), jnp.int32),
        pltpu.VMEM((WL, ROWS_W), jnp.int32),
        pltpu.VMEM((WL, ROWS_W), jnp.int32),
        pltpu.VMEM((PER_W, D), jnp.float32),
        pltpu.VMEM((PER_W, D), jnp.float32),
        pltpu.VMEM((PER_W, D), jnp.float32),
        pltpu.SemaphoreType.DMA,
    ]
    k = pl.kernel(_sc_gather_body, out_type=out_t, mesh=mesh,
                  scratch_types=scratch,
                  compiler_params=pltpu.CompilerParams(
                      use_tc_tiling_on_sc=False))
    return k(emb_word, emb_pref, emb_suff, widx, pidx, sidx)


def _mlp_body(hw, hp, hs, w1, b1, w2, b2, out):
    acc = jnp.zeros((BB, H), dtype=jnp.float32) + b1[...]
    for w in range(WL):
        avg = (hw[w] + hp[w] + hs[w]) * (1.0 / 3.0)
        acc = acc + jnp.dot(avg, w1[w * D:(w + 1) * D, :],
                            preferred_element_type=jnp.float32)
    h2 = jnp.tanh(acc)
    o = jnp.dot(h2, w2[...], preferred_element_type=jnp.float32) + b2[...]
    m = jnp.max(o, axis=1, keepdims=True)
    s = o - m
    lse = jnp.log(jnp.sum(jnp.exp(s), axis=1, keepdims=True))
    out[...] = s - lse


def _mlp(hw, hp, hs, W1, b1, W2, b2, *, interpret=False):
    x_spec = pl.BlockSpec((WL, BB, D), lambda i: (0, i, 0))
    return pl.pallas_call(
        _mlp_body,
        grid=(B // BB,),
        in_specs=[
            x_spec, x_spec, x_spec,
            pl.BlockSpec((WL * D, H), lambda i: (0, 0)),
            pl.BlockSpec((1, H), lambda i: (0, 0)),
            pl.BlockSpec((H, T), lambda i: (0, 0)),
            pl.BlockSpec((1, T), lambda i: (0, 0)),
        ],
        out_specs=pl.BlockSpec((BB, T), lambda i: (i, 0)),
        out_shape=jax.ShapeDtypeStruct((B, T), jnp.float32),
        interpret=interpret,
    )(hw, hp, hs, W1, b1.reshape(1, H), W2, b2.reshape(1, T))


def kernel(words, suffix, prefix, emb_word, emb_pref, emb_suff, W1, b1, W2, b2):
    widx, sidx, pidx = _idx_prep(words, suffix, prefix)
    hw, hp, hs = _sc_gather(emb_word, emb_pref, emb_suff, widx, pidx, sidx)
    hw = hw.reshape(WL, B, D)
    hp = hp.reshape(WL, B, D)
    hs = hs.reshape(WL, B, D)
    return _mlp(hw, hp, hs, W1, b1, W2, b2)


# speedup vs baseline: 1.0862x; 1.0862x over previous
import jax
import jax.numpy as jnp
from jax import lax
from jax.experimental import pallas as pl
from jax.experimental.pallas import tpu as pltpu
from jax.experimental.pallas import tpu_sc as plsc

B = 4096
WL = 5
D = 64
DP = 128
H = 128
T = 50
NI = B * WL

NC, NS = 2, 16
NW = NC * NS
ROWS_W = B // NW

BB = 512


def _sc_gather_body(ew, ep, es, wi, pi, si, ow, op_, os_,
                    wv2, pv2, sv2, wv, pv, sv, rw, pb, sb, sem):
    wid = lax.axis_index("s") * NC + lax.axis_index("c")
    b0 = wid * ROWS_W
    rsl2 = pl.ds(b0, ROWS_W)
    idx_cps = [
        pltpu.async_copy(wi.at[rsl2, :], wv2, sem),
        pltpu.async_copy(pi.at[rsl2, :], pv2, sem),
        pltpu.async_copy(si.at[rsl2, :], sv2, sem),
    ]
    for cp in idx_cps:
        cp.wait()
    for w in range(WL):
        cols = jnp.full((16,), w, dtype=jnp.int32)
        for j in range(ROWS_W // 16):
            rows = jnp.arange(16, dtype=jnp.int32) + (16 * j)
            lsl = pl.ds(j * 16, 16)
            wv[w, lsl] = plsc.load_gather(wv2, [rows, cols])
            pv[w, lsl] = plsc.load_gather(pv2, [rows, cols])
            sv[w, lsl] = plsc.load_gather(sv2, [rows, cols])
    word_cps = []
    for w in range(WL):
        rsl = pl.ds(w * ROWS_W, ROWS_W)
        word_cps.append(pltpu.async_copy(ew.at[wv.at[w]], rw.at[rsl], sem))
    ps_write_cps = []
    for w in range(WL):
        if w >= 2:
            ps_write_cps[2 * (w - 2)].wait()
            ps_write_cps[2 * (w - 2) + 1].wait()
        pltpu.async_copy(ep.at[pv.at[w]], pb.at[w % 2], sem).wait()
        pltpu.async_copy(es.at[sv.at[w]], sb.at[w % 2], sem).wait()
        osl = pl.ds(w * B + b0, ROWS_W)
        ps_write_cps.append(pltpu.async_copy(pb.at[w % 2], op_.at[osl], sem))
        ps_write_cps.append(pltpu.async_copy(sb.at[w % 2], os_.at[osl], sem))
    for cp in word_cps:
        cp.wait()
    out_cps = []
    for w in range(WL):
        rsl = pl.ds(w * ROWS_W, ROWS_W)
        osl = pl.ds(w * B + b0, ROWS_W)
        out_cps.append(pltpu.async_copy(rw.at[rsl], ow.at[osl], sem))
    for cp in ps_write_cps[-4:]:
        cp.wait()
    for cp in out_cps:
        cp.wait()


def _sc_gather(emb_word, emb_pref_pad, emb_suff_pad, words, prefix, suffix):
    mesh = plsc.VectorSubcoreMesh(core_axis_name="c", subcore_axis_name="s")
    out_t = [
        jax.ShapeDtypeStruct((NI, D), jnp.float32),
        jax.ShapeDtypeStruct((NI, DP), jnp.float32),
        jax.ShapeDtypeStruct((NI, DP), jnp.float32),
    ]
    scratch = [
        pltpu.VMEM((ROWS_W, WL), jnp.int32),
        pltpu.VMEM((ROWS_W, WL), jnp.int32),
        pltpu.VMEM((ROWS_W, WL), jnp.int32),
        pltpu.VMEM((WL, ROWS_W), jnp.int32),
        pltpu.VMEM((WL, ROWS_W), jnp.int32),
        pltpu.VMEM((WL, ROWS_W), jnp.int32),
        pltpu.VMEM((WL * ROWS_W, D), jnp.float32),
        pltpu.VMEM((2, ROWS_W, DP), jnp.float32),
        pltpu.VMEM((2, ROWS_W, DP), jnp.float32),
        pltpu.SemaphoreType.DMA,
    ]
    k = pl.kernel(_sc_gather_body, out_type=out_t, mesh=mesh,
                  scratch_types=scratch,
                  compiler_params=pltpu.CompilerParams(
                      use_tc_tiling_on_sc=False,
                      needs_layout_passes=False))
    return k(emb_word, emb_pref_pad, emb_suff_pad, words, prefix, suffix)


def _pad_tables_body(ep, es, epo, eso):
    epo[:, :D] = ep[...]
    epo[:, D:] = jnp.zeros_like(epo[:, D:])
    eso[:, :D] = es[...]
    eso[:, D:] = jnp.zeros_like(eso[:, D:])


def _pad_tables(emb_pref, emb_suff):
    n = emb_pref.shape[0]
    return pl.pallas_call(
        _pad_tables_body,
        out_shape=[jax.ShapeDtypeStruct((n, DP), jnp.float32)] * 2,
    )(emb_pref, emb_suff)


def _mlp_body(hw, hp, hs, w1, b1, w2, b2, out):
    acc = jnp.zeros((BB, H), dtype=jnp.float32) + b1[...]
    for w in range(WL):
        avg = (hw[w] + hp[w][:, :D] + hs[w][:, :D]) * (1.0 / 3.0)
        acc = acc + jnp.dot(avg, w1[w * D:(w + 1) * D, :],
                            preferred_element_type=jnp.float32)
    h2 = jnp.tanh(acc)
    o = jnp.dot(h2, w2[...], preferred_element_type=jnp.float32) + b2[...]
    m = jnp.max(o, axis=1, keepdims=True)
    s = o - m
    lse = jnp.log(jnp.sum(jnp.exp(s), axis=1, keepdims=True))
    out[...] = s - lse


def _mlp(hw, hp, hs, W1, b1, W2, b2, *, interpret=False):
    return pl.pallas_call(
        _mlp_body,
        grid=(B // BB,),
        in_specs=[
            pl.BlockSpec((WL, BB, D), lambda i: (0, i, 0)),
            pl.BlockSpec((WL, BB, DP), lambda i: (0, i, 0)),
            pl.BlockSpec((WL, BB, DP), lambda i: (0, i, 0)),
            pl.BlockSpec((WL * D, H), lambda i: (0, 0)),
            pl.BlockSpec((1, H), lambda i: (0, 0)),
            pl.BlockSpec((H, T), lambda i: (0, 0)),
            pl.BlockSpec((1, T), lambda i: (0, 0)),
        ],
        out_specs=pl.BlockSpec((BB, T), lambda i: (i, 0)),
        out_shape=jax.ShapeDtypeStruct((B, T), jnp.float32),
        interpret=interpret,
    )(hw, hp, hs, W1, b1.reshape(1, H), W2, b2.reshape(1, T))


def kernel(words, suffix, prefix, emb_word, emb_pref, emb_suff, W1, b1, W2, b2):
    ppad, spad = _pad_tables(emb_pref, emb_suff)
    hw, hp, hs = _sc_gather(emb_word, ppad, spad, words, prefix, suffix)
    hw = hw.reshape(WL, B, D)
    hp = hp.reshape(WL, B, DP)
    hs = hs.reshape(WL, B, DP)
    return _mlp(hw, hp, hs, W1, b1, W2, b2)
